# Initial kernel scaffold; baseline (speedup 1.0000x reference)
import jax
import jax.numpy as jnp
from jax.experimental import pallas as pl


def kernel(inputs, embed_table, W_f, U_f, b_f, W_b, U_b, b_b, W1, b1, gamma, beta, mv_mean, mv_var, W2, b2):
    raise NotImplementedError("write your pallas kernel here")


# speedup vs baseline: 6.0439x; 6.0439x over previous
import functools

import jax
import jax.numpy as jnp
from jax import lax
from jax.experimental import pallas as pl
from jax.experimental.pallas import tpu as pltpu
from jax.experimental.pallas import tpu_sc as plsc

T = 50
D = 32
H = 64
C = 10


def _make_sc_gather(n_rows, d):
    info = plsc.get_sparse_core_info()
    nw = info.num_cores * info.num_subcores
    assert n_rows % nw == 0
    per_w = n_rows // nw
    n_chunks = 4
    assert per_w % n_chunks == 0
    k = per_w // n_chunks

    mesh = plsc.VectorSubcoreMesh(core_axis_name="c", subcore_axis_name="s")

    @functools.partial(
        pl.kernel,
        mesh=mesh,
        compiler_params=pltpu.CompilerParams(use_tc_tiling_on_sc=False),
        out_type=jax.ShapeDtypeStruct((n_rows, d), jnp.float32),
        scratch_types=[
            pltpu.VMEM((per_w,), jnp.int32),
            pltpu.VMEM((k, d), jnp.float32),
            pltpu.VMEM((k, d), jnp.float32),
            pltpu.SemaphoreType.DMA,
            pltpu.SemaphoreType.DMA,
        ],
    )
    def gather(table_hbm, idx_hbm, out_hbm, idx_v, buf0, buf1, sem0, sem1):
        wid = lax.axis_index("s") * info.num_cores + lax.axis_index("c")
        base = wid * per_w
        pltpu.sync_copy(idx_hbm.at[pl.ds(base, per_w)], idx_v)
        bufs = (buf0, buf1)
        sems = (sem0, sem1)
        cp0 = pltpu.async_copy(table_hbm.at[idx_v.at[pl.ds(0, k)]], buf0, sem0)
        copies = [cp0, None]
        for c in range(n_chunks):
            nxt = c + 1
            if nxt < n_chunks:
                copies[nxt % 2] = pltpu.async_copy(
                    table_hbm.at[idx_v.at[pl.ds(nxt * k, k)]],
                    bufs[nxt % 2], sems[nxt % 2])
            copies[c % 2].wait()
            pltpu.sync_copy(bufs[c % 2], out_hbm.at[pl.ds(base + c * k, k)])

    return gather


def _lstm_step(x, h, c, W, U, b, m):
    z = (jnp.dot(x, W, preferred_element_type=jnp.float32)
         + jnp.dot(h, U, preferred_element_type=jnp.float32) + b)
    zi = z[:, 0 * H:1 * H]
    zf = z[:, 1 * H:2 * H]
    zg = z[:, 2 * H:3 * H]
    zo = z[:, 3 * H:4 * H]
    i = jax.nn.sigmoid(zi)
    f = jax.nn.sigmoid(zf)
    g = jnp.tanh(zg)
    o = jax.nn.sigmoid(zo)
    c_new = f * c + i * g
    h_new = o * jnp.tanh(c_new)
    h = jnp.where(m, h_new, h)
    c = jnp.where(m, c_new, c)
    return h, c


def _bilstm_body(tok_f_ref, tok_b_ref, emb_f_ref, emb_b_ref,
                 Wf_ref, Uf_ref, bf_ref, Wb_ref, Ub_ref, bb_ref,
                 W1_ref, b1_ref, gamma_ref, beta_ref, mean_ref, var_ref,
                 W2_ref, b2_ref,
                 out_ref, hf, cf, hb, cb):
    t = pl.program_id(0)

    @pl.when(t == 0)
    def _init():
        hf[...] = jnp.zeros_like(hf)
        cf[...] = jnp.zeros_like(cf)
        hb[...] = jnp.zeros_like(hb)
        cb[...] = jnp.zeros_like(cb)

    mf = tok_f_ref[0] != 0
    mb = tok_b_ref[0] != 0

    h, c = _lstm_step(emb_f_ref[0], hf[...], cf[...],
                      Wf_ref[...], Uf_ref[...], bf_ref[...], mf)
    hf[...] = h
    cf[...] = c
    h, c = _lstm_step(emb_b_ref[0], hb[...], cb[...],
                      Wb_ref[...], Ub_ref[...], bb_ref[...], mb)
    hb[...] = h
    cb[...] = c

    @pl.when(t == T - 1)
    def _head():
        hcat = jnp.concatenate([hf[...], hb[...]], axis=1)
        x = jnp.dot(hcat, W1_ref[...], preferred_element_type=jnp.float32)
        x = jax.nn.relu(x + b1_ref[...])
        x = (gamma_ref[...] * (x - mean_ref[...])
             / jnp.sqrt(var_ref[...] + 1e-3) + beta_ref[...])
        logits = jnp.dot(x, W2_ref[...], preferred_element_type=jnp.float32)
        logits = logits + b2_ref[...]
        logits = logits - jnp.max(logits, axis=-1, keepdims=True)
        e = jnp.exp(logits)
        out_ref[...] = e / jnp.sum(e, axis=-1, keepdims=True)


def _bilstm_head(tokens, emb_tbd, Wf, Uf, bf, Wb, Ub, bb,
                 W1, b1, gamma, beta, mean, var, W2, b2, interpret=False):
    B = tokens.shape[0]
    row = lambda v: v.reshape(1, -1)
    grid = (T,)
    full = lambda shp: pl.BlockSpec(shp, lambda t: (0,) * len(shp))
    call = pl.pallas_call(
        _bilstm_body,
        grid=grid,
        in_specs=[
            pl.BlockSpec((1, B, 1), lambda t: (t, 0, 0)),
            pl.BlockSpec((1, B, 1), lambda t: (T - 1 - t, 0, 0)),
            pl.BlockSpec((1, B, D), lambda t: (t, 0, 0)),
            pl.BlockSpec((1, B, D), lambda t: (T - 1 - t, 0, 0)),
            full((D, 4 * H)), full((H, 4 * H)), full((1, 4 * H)),
            full((D, 4 * H)), full((H, 4 * H)), full((1, 4 * H)),
            full((2 * H, H)), full((1, H)),
            full((1, H)), full((1, H)), full((1, H)), full((1, H)),
            full((H, C)), full((1, C)),
        ],
        out_specs=pl.BlockSpec((B, C), lambda t: (0, 0)),
        out_shape=jax.ShapeDtypeStruct((B, C), jnp.float32),
        scratch_shapes=[pltpu.VMEM((B, H), jnp.float32)] * 4,
        interpret=interpret,
    )
    tokens_tb1 = tokens.T[:, :, None]
    return call(tokens_tb1, tokens_tb1, emb_tbd, emb_tbd,
                Wf, Uf, row(bf), Wb, Ub, row(bb),
                W1, row(b1), row(gamma), row(beta), row(mean), row(var),
                W2, row(b2))


def kernel(inputs, embed_table, W_f, U_f, b_f, W_b, U_b, b_b,
           W1, b1, gamma, beta, mv_mean, mv_var, W2, b2):
    B = inputs.shape[0]
    tokens = inputs.astype(jnp.int32)
    idx_tmajor = tokens.T.reshape(-1)
    gather = _make_sc_gather(T * B, D)
    emb = gather(embed_table, idx_tmajor).reshape(T, B, D)
    return _bilstm_head(tokens, emb, W_f, U_f, b_f, W_b, U_b, b_b,
                        W1, b1, gamma, beta, mv_mean, mv_var, W2, b2)


# speedup vs baseline: 6.1612x; 1.0194x over previous
import functools

import jax
import jax.numpy as jnp
from jax import lax
from jax.experimental import pallas as pl
from jax.experimental.pallas import tpu as pltpu
from jax.experimental.pallas import tpu_sc as plsc

T = 50
D = 32
H = 64
C = 10


def _make_sc_gather(n_rows, d):
    info = plsc.get_sparse_core_info()
    nw = info.num_cores * info.num_subcores
    assert n_rows % nw == 0
    per_w = n_rows // nw
    n_chunks = 4
    assert per_w % n_chunks == 0
    k = per_w // n_chunks

    mesh = plsc.VectorSubcoreMesh(core_axis_name="c", subcore_axis_name="s")

    @functools.partial(
        pl.kernel,
        mesh=mesh,
        compiler_params=pltpu.CompilerParams(use_tc_tiling_on_sc=False),
        out_type=jax.ShapeDtypeStruct((n_rows, d), jnp.float32),
        scratch_types=[
            pltpu.VMEM((per_w,), jnp.int32),
            pltpu.VMEM((k, d), jnp.float32),
            pltpu.VMEM((k, d), jnp.float32),
            pltpu.SemaphoreType.DMA,
            pltpu.SemaphoreType.DMA,
        ],
    )
    def gather(table_hbm, idx_hbm, out_hbm, idx_v, buf0, buf1, sem0, sem1):
        wid = lax.axis_index("s") * info.num_cores + lax.axis_index("c")
        base = wid * per_w
        pltpu.sync_copy(idx_hbm.at[pl.ds(base, per_w)], idx_v)
        bufs = (buf0, buf1)
        sems = (sem0, sem1)
        cp0 = pltpu.async_copy(table_hbm.at[idx_v.at[pl.ds(0, k)]], buf0, sem0)
        copies = [cp0, None]
        for c in range(n_chunks):
            nxt = c + 1
            if nxt < n_chunks:
                copies[nxt % 2] = pltpu.async_copy(
                    table_hbm.at[idx_v.at[pl.ds(nxt * k, k)]],
                    bufs[nxt % 2], sems[nxt % 2])
            copies[c % 2].wait()
            pltpu.sync_copy(bufs[c % 2], out_hbm.at[pl.ds(base + c * k, k)])

    return gather


def _sigmoid(x):
    return 0.5 * jnp.tanh(0.5 * x) + 0.5


def _lstm_step(x, h, c, W, U, b, m):
    bf = jnp.bfloat16
    z = (jnp.dot(x.astype(bf), W, preferred_element_type=jnp.float32)
         + jnp.dot(h.astype(bf), U, preferred_element_type=jnp.float32) + b)
    zi = z[:, 0 * H:1 * H]
    zf = z[:, 1 * H:2 * H]
    zg = z[:, 2 * H:3 * H]
    zo = z[:, 3 * H:4 * H]
    i = _sigmoid(zi)
    f = _sigmoid(zf)
    g = jnp.tanh(zg)
    o = _sigmoid(zo)
    c_new = f * c + i * g
    h_new = o * jnp.tanh(c_new)
    m64 = jnp.broadcast_to(m, h.shape)
    h = jnp.where(m64, h_new, h)
    c = jnp.where(m64, c_new, c)
    return h, c


def _bilstm_body(tok_f_ref, tok_b_ref, emb_f_ref, emb_b_ref,
                 Wf_ref, Uf_ref, bf_ref, Wb_ref, Ub_ref, bb_ref,
                 W1_ref, b1_ref, gamma_ref, beta_ref, mean_ref, var_ref,
                 W2_ref, b2_ref,
                 out_ref, hf, cf, hb, cb):
    t = pl.program_id(0)

    @pl.when(t == 0)
    def _init():
        hf[...] = jnp.zeros_like(hf)
        cf[...] = jnp.zeros_like(cf)
        hb[...] = jnp.zeros_like(hb)
        cb[...] = jnp.zeros_like(cb)

    mf = tok_f_ref[0] != 0
    mb = tok_b_ref[0] != 0

    h, c = _lstm_step(emb_f_ref[0], hf[...], cf[...],
                      Wf_ref[...], Uf_ref[...], bf_ref[...], mf)
    hf[...] = h
    cf[...] = c
    h, c = _lstm_step(emb_b_ref[0], hb[...], cb[...],
                      Wb_ref[...], Ub_ref[...], bb_ref[...], mb)
    hb[...] = h
    cb[...] = c

    @pl.when(t == T - 1)
    def _head():
        hcat = jnp.concatenate([hf[...], hb[...]], axis=1)
        x = jnp.dot(hcat, W1_ref[...], preferred_element_type=jnp.float32)
        x = jax.nn.relu(x + b1_ref[...])
        x = (gamma_ref[...] * (x - mean_ref[...])
             / jnp.sqrt(var_ref[...] + 1e-3) + beta_ref[...])
        logits = jnp.dot(x, W2_ref[...], preferred_element_type=jnp.float32)
        logits = logits + b2_ref[...]
        logits = logits - jnp.max(logits, axis=-1, keepdims=True)
        e = jnp.exp(logits)
        out_ref[...] = e / jnp.sum(e, axis=-1, keepdims=True)


def _bilstm_head(tokens, emb_tbd, Wf, Uf, bf, Wb, Ub, bb,
                 W1, b1, gamma, beta, mean, var, W2, b2, interpret=False):
    B = tokens.shape[0]
    row = lambda v: v.reshape(1, -1)
    grid = (T,)
    full = lambda shp: pl.BlockSpec(shp, lambda t: (0,) * len(shp))
    call = pl.pallas_call(
        _bilstm_body,
        grid=grid,
        in_specs=[
            pl.BlockSpec((1, B, 1), lambda t: (t, 0, 0)),
            pl.BlockSpec((1, B, 1), lambda t: (T - 1 - t, 0, 0)),
            pl.BlockSpec((1, B, D), lambda t: (t, 0, 0)),
            pl.BlockSpec((1, B, D), lambda t: (T - 1 - t, 0, 0)),
            full((D, 4 * H)), full((H, 4 * H)), full((1, 4 * H)),
            full((D, 4 * H)), full((H, 4 * H)), full((1, 4 * H)),
            full((2 * H, H)), full((1, H)),
            full((1, H)), full((1, H)), full((1, H)), full((1, H)),
            full((H, C)), full((1, C)),
        ],
        out_specs=pl.BlockSpec((B, C), lambda t: (0, 0)),
        out_shape=jax.ShapeDtypeStruct((B, C), jnp.float32),
        scratch_shapes=[pltpu.VMEM((B, H), jnp.float32)] * 4,
        interpret=interpret,
    )
    tokens_tb1 = tokens.T[:, :, None]
    h16 = jnp.bfloat16
    return call(tokens_tb1, tokens_tb1, emb_tbd, emb_tbd,
                Wf.astype(h16), Uf.astype(h16), row(bf),
                Wb.astype(h16), Ub.astype(h16), row(bb),
                W1, row(b1), row(gamma), row(beta), row(mean), row(var),
                W2, row(b2))


def kernel(inputs, embed_table, W_f, U_f, b_f, W_b, U_b, b_b,
           W1, b1, gamma, beta, mv_mean, mv_var, W2, b2):
    B = inputs.shape[0]
    tokens = inputs.astype(jnp.int32)
    idx_tmajor = tokens.T.reshape(-1)
    gather = _make_sc_gather(T * B, D)
    emb = gather(embed_table, idx_tmajor).reshape(T, B, D)
    return _bilstm_head(tokens, emb, W_f, U_f, b_f, W_b, U_b, b_b,
                        W1, b1, gamma, beta, mv_mean, mv_var, W2, b2)


# speedup vs baseline: 8.3044x; 1.3478x over previous
import functools

import jax
import jax.numpy as jnp
from jax import lax
from jax.experimental import pallas as pl
from jax.experimental.pallas import tpu as pltpu
from jax.experimental.pallas import tpu_sc as plsc

T = 50
D = 32
H = 64
C = 10
PK = 4
LANES = 128


def _make_sc_gather(B):
    info = plsc.get_sparse_core_info()
    nw = info.num_cores * info.num_subcores
    assert B % (nw * PK) == 0
    bw = B // nw
    t_chunk = 10
    n_chunks = T // t_chunk
    k = bw * t_chunk

    mesh = plsc.VectorSubcoreMesh(core_axis_name="c", subcore_axis_name="s")

    @functools.partial(
        pl.kernel,
        mesh=mesh,
        compiler_params=pltpu.CompilerParams(use_tc_tiling_on_sc=False),
        out_type=jax.ShapeDtypeStruct((T * B, D), jnp.float32),
        scratch_types=[
            pltpu.VMEM((T * bw,), jnp.int32),
            pltpu.VMEM((k, D), jnp.float32),
            pltpu.VMEM((k, D), jnp.float32),
            pltpu.SemaphoreType.DMA,
            pltpu.SemaphoreType.DMA,
            pltpu.SemaphoreType.DMA,
            pltpu.SemaphoreType.DMA,
        ],
    )
    def gather(table_hbm, idx_hbm, emb_hbm,
               idx_v, buf0, buf1, isem, gsem0, gsem1, ssem):
        wid = lax.axis_index("s") * info.num_cores + lax.axis_index("c")

        idx_cps = [
            pltpu.async_copy(idx_hbm.at[t, pl.ds(wid * bw, bw)],
                             idx_v.at[pl.ds(t * bw, bw)], isem)
            for t in range(T)
        ]
        for cp in idx_cps:
            cp.wait()

        bufs = (buf0, buf1)
        gsems = (gsem0, gsem1)

        def fire(c):
            return pltpu.async_copy(
                table_hbm.at[idx_v.at[pl.ds(c * k, k)]],
                bufs[c % 2], gsems[c % 2])

        pend = [None, None]
        pend[0] = fire(0)
        for c in range(n_chunks):
            if c + 1 < n_chunks:
                pend[(c + 1) % 2] = fire(c + 1)
            pend[c % 2].wait()
            buf = bufs[c % 2]
            scat = [
                pltpu.async_copy(
                    buf.at[pl.ds(tl * bw, bw), :],
                    emb_hbm.at[pl.ds((c * t_chunk + tl) * B + wid * bw,
                                     bw), :],
                    ssem)
                for tl in range(t_chunk)
            ]
            for cp in scat:
                cp.wait()

    return gather, nw


def _sigmoid(x):
    return 0.5 * jnp.tanh(0.5 * x) + 0.5


def _step_packed(x4, h, c, m, W4p, U4p, b4p):
    G = PK * H
    bf = jnp.bfloat16
    z = (jnp.dot(x4.astype(bf), W4p, preferred_element_type=jnp.float32)
         + jnp.dot(h.astype(bf), U4p, preferred_element_type=jnp.float32)
         + b4p)
    i = _sigmoid(z[:, 0 * G:1 * G])
    f = _sigmoid(z[:, 1 * G:2 * G])
    g = jnp.tanh(z[:, 2 * G:3 * G])
    o = _sigmoid(z[:, 3 * G:4 * G])
    c_new = f * c + i * g
    h_new = o * jnp.tanh(c_new)
    h = h + m * (h_new - h)
    c = c + m * (c_new - c)
    return h, c


def _mask_packed(mref, B4):
    tok = mref[0]
    m4 = jnp.where(tok != 0, jnp.float32(1.0), jnp.float32(0.0))
    sel = (lax.broadcasted_iota(jnp.int32, (PK, PK * H), 1) // H ==
           lax.broadcasted_iota(jnp.int32, (PK, PK * H), 0)
           ).astype(jnp.float32)
    return jnp.dot(m4, sel, preferred_element_type=jnp.float32)


def _packed_body(mk_f_ref, mk_b_ref, emb_f_ref, emb_b_ref,
                 Wf_ref, Uf_ref, bf_ref, Wb_ref, Ub_ref, bb_ref,
                 W1_ref, b1_ref, bns_ref, bnb_ref, W2_ref, b2_ref,
                 out_ref, hf, cf, hb, cb):
    t = pl.program_id(0)
    B4 = hf.shape[0]
    G = PK * H

    @pl.when(t == 0)
    def _init():
        hf[...] = jnp.zeros_like(hf)
        cf[...] = jnp.zeros_like(cf)
        hb[...] = jnp.zeros_like(hb)
        cb[...] = jnp.zeros_like(cb)

    mf = _mask_packed(mk_f_ref, B4)
    mb = _mask_packed(mk_b_ref, B4)

    h, c = _step_packed(emb_f_ref[0], hf[...], cf[...], mf,
                        Wf_ref[...], Uf_ref[...], bf_ref[...])
    hf[...] = h
    cf[...] = c
    h, c = _step_packed(emb_b_ref[0], hb[...], cb[...], mb,
                        Wb_ref[...], Ub_ref[...], bb_ref[...])
    hb[...] = h
    cb[...] = c

    @pl.when(t == T - 1)
    def _head():
        hcat = jnp.concatenate([hf[...], hb[...]], axis=1)
        x = jnp.dot(hcat, W1_ref[...], preferred_element_type=jnp.float32)
        x = jax.nn.relu(x + b1_ref[...])
        x = x * bns_ref[...] + bnb_ref[...]
        logits = jnp.dot(x, W2_ref[...], preferred_element_type=jnp.float32)
        logits = (logits + b2_ref[...]).reshape(B4, PK, C)
        logits = logits - jnp.max(logits, axis=-1, keepdims=True)
        e = jnp.exp(logits)
        out_ref[...] = e / jnp.sum(e, axis=-1, keepdims=True)


def _pack_in(Wg, n_in):
    Wr = Wg.reshape(n_in, 4, H)
    eye = jnp.eye(PK, dtype=Wg.dtype)
    return jnp.einsum("dgk,jl->jdglk", Wr, eye).reshape(PK * n_in, 4 * PK * H)


def _bilstm_packed(emb4, tokp4, nw, Wf, Uf, bf, Wb, Ub, bb,
                   W1, b1, gamma, beta, mean, var, W2, b2, interpret=False):
    B4, NW = emb4.shape[1], nw
    B = B4 * PK
    G = PK * H
    h16 = jnp.bfloat16
    eye = jnp.eye(PK, dtype=jnp.float32)

    W4f = _pack_in(Wf, D).astype(h16)
    U4f = _pack_in(Uf, H).astype(h16)
    W4b = _pack_in(Wb, D).astype(h16)
    U4b = _pack_in(Ub, H).astype(h16)
    b4f = jnp.tile(bf.reshape(4, 1, H), (1, PK, 1)).reshape(1, 4 * G)
    b4b = jnp.tile(bb.reshape(4, 1, H), (1, PK, 1)).reshape(1, 4 * G)

    W1p = jnp.einsum("duv,jl->djulv", W1.reshape(2, H, H), eye).reshape(2 * G, G)
    b1p = jnp.tile(b1, PK).reshape(1, G)
    scale = gamma / jnp.sqrt(var + 1e-3)
    bias = beta - mean * scale
    bnsp = jnp.tile(scale, PK).reshape(1, G)
    bnbp = jnp.tile(bias, PK).reshape(1, G)
    W2p = jnp.einsum("vc,jl->jvlc", W2, eye).reshape(G, PK * C)
    b2p = jnp.tile(b2, PK).reshape(1, PK * C)

    full = lambda shp: pl.BlockSpec(shp, lambda t: (0,) * len(shp))
    call = pl.pallas_call(
        _packed_body,
        grid=(T,),
        in_specs=[
            pl.BlockSpec((1, B4, PK), lambda t: (t, 0, 0)),
            pl.BlockSpec((1, B4, PK), lambda t: (T - 1 - t, 0, 0)),
            pl.BlockSpec((1, B4, LANES), lambda t: (t, 0, 0)),
            pl.BlockSpec((1, B4, LANES), lambda t: (T - 1 - t, 0, 0)),
            full((PK * D, 4 * G)), full((PK * H, 4 * G)), full((1, 4 * G)),
            full((PK * D, 4 * G)), full((PK * H, 4 * G)), full((1, 4 * G)),
            full((2 * G, G)), full((1, G)), full((1, G)), full((1, G)),
            full((G, PK * C)), full((1, PK * C)),
        ],
        out_specs=pl.BlockSpec((B4, PK, C), lambda t: (0, 0, 0)),
        out_shape=jax.ShapeDtypeStruct((B4, PK, C), jnp.float32),
        scratch_shapes=[pltpu.VMEM((B4, G), jnp.float32)] * 4,
        interpret=interpret,
    )
    out4 = call(tokp4, tokp4, emb4, emb4,
                W4f, U4f, b4f, W4b, U4b, b4b,
                W1p, b1p, bnsp, bnbp, W2p, b2p)
    rw = B4 // NW
    return (out4.reshape(NW, rw, PK, C)
            .transpose(0, 2, 1, 3).reshape(B, C))


def _perm_indices(tokens, nw):
    B = tokens.shape[0]
    bw = B // nw
    rw = bw // PK
    q = jnp.arange(B, dtype=jnp.int32)
    p, j = q // PK, q % PK
    b = bw * (p // rw) + rw * j + (p % rw)
    return jnp.take(tokens, b, axis=0).T


def kernel(inputs, embed_table, W_f, U_f, b_f, W_b, U_b, b_b,
           W1, b1, gamma, beta, mv_mean, mv_var, W2, b2):
    B = inputs.shape[0]
    tokens = inputs.astype(jnp.int32)
    gather, nw = _make_sc_gather(B)
    idx_tb = _perm_indices(tokens, nw)
    emb2d = gather(embed_table, idx_tb)
    emb4 = emb2d.reshape(T, B // PK, LANES)
    tokp4 = idx_tb.reshape(T, B // PK, PK)
    return _bilstm_packed(emb4, tokp4, nw, W_f, U_f, b_f, W_b, U_b, b_b,
                          W1, b1, gamma, beta, mv_mean, mv_var, W2, b2)
